# Initial kernel scaffold; baseline (speedup 1.0000x reference)
import jax
import jax.numpy as jnp
from jax.experimental import pallas as pl


def kernel(z, edge_index, W1, b1, g1, bb1, W2, b2, g2, bb2, mW1, mb1, mW2, mb2):
    raise NotImplementedError("write your pallas kernel here")


# speedup vs baseline: 14.3701x; 14.3701x over previous
import jax
import jax.numpy as jnp
from jax import lax
from jax.experimental import pallas as pl
from jax.experimental.pallas import tpu as pltpu
from jax.experimental.pallas import tpu_sc as plsc

N = 10000
D = 128
H = 128
C = 3
E = 320000

NC = 2
NS = 16
NW = NC * NS
HH = H // NC
CHUNK = 80

EPT = E // NS
NCH_P = EPT // CHUNK
EPW = E // NW
NCH_D = EPW // CHUNK

NPAD = 10240
RPT = NPAD // NS
ZROWS = 128
DEGW = 16

_mesh = plsc.VectorSubcoreMesh(
    core_axis_name="c", subcore_axis_name="s", num_cores=NC, num_subcores=NS
)


def _sc_deg_body(dst_hbm, deg_out, dstv, onesv, zv, deg_sh):
    c = lax.axis_index("c")
    s = lax.axis_index("s")
    wid = s * NC + c

    def fill_zero(i, carry):
        zv[i, :] = jnp.zeros((16,), jnp.float32)
        return carry

    lax.fori_loop(0, RPT, fill_zero, 0)

    def fill_one(i, carry):
        onesv[i, :] = jnp.ones((16,), jnp.float32)
        return carry

    lax.fori_loop(0, CHUNK, fill_one, 0)

    pltpu.sync_copy(zv, deg_sh.at[pl.ds(s * RPT, RPT)])
    plsc.subcore_barrier()

    pltpu.sync_copy(dst_hbm.at[wid], dstv)

    def step(ci, carry):
        pltpu.sync_copy(onesv, deg_sh.at[dstv.at[ci]], add=True)
        return carry

    lax.fori_loop(0, NCH_D, step, 0)
    plsc.subcore_barrier()

    pltpu.sync_copy(
        deg_sh.at[pl.ds(s * RPT, RPT)], deg_out.at[c, pl.ds(s * RPT, RPT)]
    )


_sc_deg = pl.kernel(
    _sc_deg_body,
    out_type=jax.ShapeDtypeStruct((NC, NPAD, DEGW), jnp.float32),
    mesh=_mesh,
    scratch_types=[
        pltpu.VMEM((NCH_D, CHUNK), jnp.int32),
        pltpu.VMEM((CHUNK, DEGW), jnp.float32),
        pltpu.VMEM((RPT, DEGW), jnp.float32),
        pltpu.VMEM_SHARED((NPAD, DEGW), jnp.float32),
    ],
    compiler_params=pltpu.CompilerParams(use_tc_tiling_on_sc=False),
)


def _sc_pass_body(hs_hbm, src_hbm, dst_hbm, acc_out, srcv, dstv, rows, zb, acc_sh, gsem):
    c = lax.axis_index("c")
    s = lax.axis_index("s")

    def fill_zero(i, carry):
        for j in range(HH // 16):
            zb[i, pl.ds(j * 16, 16)] = jnp.zeros((16,), jnp.float32)
        return carry

    lax.fori_loop(0, ZROWS, fill_zero, 0)
    for k in range(RPT // ZROWS):
        pltpu.sync_copy(zb, acc_sh.at[pl.ds(s * RPT + k * ZROWS, ZROWS)])
    plsc.subcore_barrier()

    pltpu.sync_copy(src_hbm.at[s], srcv)
    pltpu.sync_copy(dst_hbm.at[s], dstv)

    def step(ci, carry):
        pltpu.async_copy(hs_hbm.at[c].at[srcv.at[ci]], rows, gsem).wait()
        pltpu.sync_copy(rows, acc_sh.at[dstv.at[ci]], add=True)
        return carry

    lax.fori_loop(0, NCH_P, step, 0)
    plsc.subcore_barrier()

    pltpu.sync_copy(
        acc_sh.at[pl.ds(s * RPT, RPT)], acc_out.at[c, pl.ds(s * RPT, RPT)]
    )


_sc_pass = pl.kernel(
    _sc_pass_body,
    out_type=jax.ShapeDtypeStruct((NC, NPAD, HH), jnp.float32),
    mesh=_mesh,
    scratch_types=[
        pltpu.VMEM((NCH_P, CHUNK), jnp.int32),
        pltpu.VMEM((NCH_P, CHUNK), jnp.int32),
        pltpu.VMEM((CHUNK, HH), jnp.float32),
        pltpu.VMEM((ZROWS, HH), jnp.float32),
        pltpu.VMEM_SHARED((NPAD, HH), jnp.float32),
        pltpu.SemaphoreType.DMA,
    ],
    compiler_params=pltpu.CompilerParams(use_tc_tiling_on_sc=False),
)


RB = 2000


def _dinv_from(deg_ref):
    deg = deg_ref[0, :, 0:1] + deg_ref[1, :, 0:1] + 1.0
    return lax.rsqrt(deg)


def _tc_prep_body(z_ref, w_ref, deg_ref, h_ref, hs_ref):
    dinv = _dinv_from(deg_ref)
    h = jnp.dot(z_ref[...], w_ref[...], preferred_element_type=jnp.float32)
    h_ref[...] = h
    hs = h * dinv
    hs_ref[0] = hs[:, :HH]
    hs_ref[1] = hs[:, HH:]


def _ln_relu_agg(acc_ref, h_ref, dinv, b_ref, g_ref, bb_ref):
    acc = jnp.concatenate([acc_ref[0], acc_ref[1]], axis=-1)
    agg = acc * dinv + h_ref[...] * (dinv * dinv) + b_ref[...]
    x = jnp.maximum(agg, 0.0)
    mu = jnp.mean(x, axis=-1, keepdims=True)
    xc = x - mu
    var = jnp.mean(xc * xc, axis=-1, keepdims=True)
    return xc * lax.rsqrt(var + 1e-5) * g_ref[...] + bb_ref[...]


def _tc_mid_body(acc_ref, h_ref, deg_ref, b_ref, g_ref, bb_ref, w_ref, h2_ref, hs2_ref):
    dinv = _dinv_from(deg_ref)
    xn = _ln_relu_agg(acc_ref, h_ref, dinv, b_ref, g_ref, bb_ref)
    h2 = jnp.dot(xn, w_ref[...], preferred_element_type=jnp.float32)
    h2_ref[...] = h2
    hs2 = h2 * dinv
    hs2_ref[0] = hs2[:, :HH]
    hs2_ref[1] = hs2[:, HH:]


def _tc_final_body(acc_ref, h_ref, deg_ref, b_ref, g_ref, bb_ref,
                   mw1_ref, mb1_ref, mw2_ref, mb2_ref, y_ref):
    dinv = _dinv_from(deg_ref)
    xn = _ln_relu_agg(acc_ref, h_ref, dinv, b_ref, g_ref, bb_ref)
    u = jnp.maximum(
        jnp.dot(xn, mw1_ref[...], preferred_element_type=jnp.float32) + mb1_ref[...],
        0.0,
    )
    y_ref[...] = (
        jnp.dot(u, mw2_ref[...], preferred_element_type=jnp.float32) + mb2_ref[...]
    )


_GRID = N // RB

_z_spec = pl.BlockSpec((RB, H), lambda i: (i, 0))
_w_spec = pl.BlockSpec((H, H), lambda i: (0, 0))
_deg_spec = pl.BlockSpec((NC, RB, DEGW), lambda i: (0, i, 0))
_acc_spec = pl.BlockSpec((NC, RB, HH), lambda i: (0, i, 0))
_vec_spec = pl.BlockSpec((1, H), lambda i: (0, 0))

_nh_struct = jax.ShapeDtypeStruct((N, H), jnp.float32)
_split_struct = jax.ShapeDtypeStruct((NC, N, HH), jnp.float32)
_split_spec = pl.BlockSpec((NC, RB, HH), lambda i: (0, i, 0))

_tc_prep = pl.pallas_call(
    _tc_prep_body,
    grid=(_GRID,),
    in_specs=[_z_spec, _w_spec, _deg_spec],
    out_specs=[_z_spec, _split_spec],
    out_shape=[_nh_struct, _split_struct],
)

_tc_mid = pl.pallas_call(
    _tc_mid_body,
    grid=(_GRID,),
    in_specs=[_acc_spec, _z_spec, _deg_spec, _vec_spec, _vec_spec, _vec_spec, _w_spec],
    out_specs=[_z_spec, _split_spec],
    out_shape=[_nh_struct, _split_struct],
)

_tc_final = pl.pallas_call(
    _tc_final_body,
    grid=(_GRID,),
    in_specs=[_acc_spec, _z_spec, _deg_spec, _vec_spec, _vec_spec, _vec_spec,
              _w_spec, _vec_spec, _w_spec, _vec_spec],
    out_specs=_z_spec,
    out_shape=_nh_struct,
)


@jax.jit
def kernel(z, edge_index, W1, b1, g1, bb1, W2, b2, g2, bb2, mW1, mb1, mW2, mb2):
    dst_d = edge_index[1].reshape(NW, NCH_D, CHUNK)
    src_p = edge_index[0].reshape(NS, NCH_P, CHUNK)
    dst_p = edge_index[1].reshape(NS, NCH_P, CHUNK)

    deg2 = _sc_deg(dst_d)
    h1, hs1 = _tc_prep(z, W1, deg2)
    acc1 = _sc_pass(hs1, src_p, dst_p)
    h2, hs2 = _tc_mid(
        acc1, h1, deg2,
        b1.reshape(1, H), g1.reshape(1, H), bb1.reshape(1, H), W2,
    )
    acc2 = _sc_pass(hs2, src_p, dst_p)

    mW2p = jnp.zeros((H, H), jnp.float32).at[:, :C].set(mW2)
    mb2p = jnp.zeros((1, H), jnp.float32).at[0, :C].set(mb2)
    y = _tc_final(
        acc2, h2, deg2,
        b2.reshape(1, H), g2.reshape(1, H), bb2.reshape(1, H),
        mW1, mb1.reshape(1, H), mW2p, mb2p,
    )
    return y[:, :C]


# speedup vs baseline: 17.3831x; 1.2097x over previous
import jax
import jax.numpy as jnp
from jax import lax
from jax.experimental import pallas as pl
from jax.experimental.pallas import tpu as pltpu
from jax.experimental.pallas import tpu_sc as plsc

N = 10000
D = 128
H = 128
C = 3
E = 320000

NC = 2
NS = 16
NW = NC * NS
HH = H // NC
CHUNK = 80

EPT = E // NS
NCH_P = EPT // CHUNK
EPW = E // NW
NCH_D = EPW // CHUNK

NPAD = 10240
RPT = NPAD // NS
ZROWS = 128
DEGW = 16

_mesh = plsc.VectorSubcoreMesh(
    core_axis_name="c", subcore_axis_name="s", num_cores=NC, num_subcores=NS
)


def _sc_deg_body(dst_hbm, deg_out, dstv, onesv, zv, deg_sh):
    c = lax.axis_index("c")
    s = lax.axis_index("s")
    wid = s * NC + c

    def fill_zero(i, carry):
        zv[i, :] = jnp.zeros((16,), jnp.float32)
        return carry

    lax.fori_loop(0, RPT, fill_zero, 0)

    def fill_one(i, carry):
        onesv[i, :] = jnp.ones((16,), jnp.float32)
        return carry

    lax.fori_loop(0, CHUNK, fill_one, 0)

    pltpu.sync_copy(zv, deg_sh.at[pl.ds(s * RPT, RPT)])
    plsc.subcore_barrier()

    pltpu.sync_copy(dst_hbm.at[wid], dstv)

    def step(ci, carry):
        pltpu.sync_copy(onesv, deg_sh.at[dstv.at[ci]], add=True)
        return carry

    lax.fori_loop(0, NCH_D, step, 0)
    plsc.subcore_barrier()

    pltpu.sync_copy(
        deg_sh.at[pl.ds(s * RPT, RPT)], deg_out.at[c, pl.ds(s * RPT, RPT)]
    )


_sc_deg = pl.kernel(
    _sc_deg_body,
    out_type=jax.ShapeDtypeStruct((NC, NPAD, DEGW), jnp.float32),
    mesh=_mesh,
    scratch_types=[
        pltpu.VMEM((NCH_D, CHUNK), jnp.int32),
        pltpu.VMEM((CHUNK, DEGW), jnp.float32),
        pltpu.VMEM((RPT, DEGW), jnp.float32),
        pltpu.VMEM_SHARED((NPAD, DEGW), jnp.float32),
    ],
    compiler_params=pltpu.CompilerParams(use_tc_tiling_on_sc=False),
)


def _sc_pass_body(hs_hbm, src_hbm, dst_hbm, acc_out,
                  srcv, dstv, rows0, rows1, zb, acc_sh, gsem):
    c = lax.axis_index("c")
    s = lax.axis_index("s")

    def fill_zero(i, carry):
        for j in range(HH // 16):
            zb[i, pl.ds(j * 16, 16)] = jnp.zeros((16,), jnp.float32)
        return carry

    lax.fori_loop(0, ZROWS, fill_zero, 0)
    for k in range(RPT // ZROWS):
        pltpu.sync_copy(zb, acc_sh.at[pl.ds(s * RPT + k * ZROWS, ZROWS)])
    plsc.subcore_barrier()

    pltpu.sync_copy(src_hbm.at[s], srcv)
    pltpu.sync_copy(dst_hbm.at[s], dstv)

    def gather(ci, buf):
        return pltpu.make_async_copy(hs_hbm.at[c].at[srcv.at[ci]], buf, gsem)

    gather(0, rows0).start()

    def step(g, carry):
        a = 2 * g
        b = a + 1
        gather(a, rows0).wait()
        gather(b, rows1).start()
        pltpu.sync_copy(rows0, acc_sh.at[dstv.at[a]], add=True)
        gather(b, rows1).wait()

        @pl.when(g + 1 < NCH_P // 2)
        def _():
            gather(a + 2, rows0).start()

        pltpu.sync_copy(rows1, acc_sh.at[dstv.at[b]], add=True)
        return carry

    lax.fori_loop(0, NCH_P // 2, step, 0)
    plsc.subcore_barrier()

    pltpu.sync_copy(
        acc_sh.at[pl.ds(s * RPT, RPT)], acc_out.at[c, pl.ds(s * RPT, RPT)]
    )


_sc_pass = pl.kernel(
    _sc_pass_body,
    out_type=jax.ShapeDtypeStruct((NC, NPAD, HH), jnp.float32),
    mesh=_mesh,
    scratch_types=[
        pltpu.VMEM((NCH_P, CHUNK), jnp.int32),
        pltpu.VMEM((NCH_P, CHUNK), jnp.int32),
        pltpu.VMEM((CHUNK, HH), jnp.float32),
        pltpu.VMEM((CHUNK, HH), jnp.float32),
        pltpu.VMEM((ZROWS, HH), jnp.float32),
        pltpu.VMEM_SHARED((NPAD, HH), jnp.float32),
        pltpu.SemaphoreType.DMA,
    ],
    compiler_params=pltpu.CompilerParams(use_tc_tiling_on_sc=False),
)


RB = 2000


def _dinv_from(deg_ref):
    deg = deg_ref[0, :, 0:1] + deg_ref[1, :, 0:1] + 1.0
    return lax.rsqrt(deg)


def _tc_prep_body(z_ref, w_ref, deg_ref, h_ref, hs_ref):
    dinv = _dinv_from(deg_ref)
    h = jnp.dot(z_ref[...], w_ref[...], preferred_element_type=jnp.float32)
    h_ref[...] = h
    hs = h * dinv
    hs_ref[0] = hs[:, :HH]
    hs_ref[1] = hs[:, HH:]


def _ln_relu_agg(acc_ref, h_ref, dinv, b_ref, g_ref, bb_ref):
    acc = jnp.concatenate([acc_ref[0], acc_ref[1]], axis=-1)
    agg = acc * dinv + h_ref[...] * (dinv * dinv) + b_ref[...]
    x = jnp.maximum(agg, 0.0)
    mu = jnp.mean(x, axis=-1, keepdims=True)
    xc = x - mu
    var = jnp.mean(xc * xc, axis=-1, keepdims=True)
    return xc * lax.rsqrt(var + 1e-5) * g_ref[...] + bb_ref[...]


def _tc_mid_body(acc_ref, h_ref, deg_ref, b_ref, g_ref, bb_ref, w_ref, h2_ref, hs2_ref):
    dinv = _dinv_from(deg_ref)
    xn = _ln_relu_agg(acc_ref, h_ref, dinv, b_ref, g_ref, bb_ref)
    h2 = jnp.dot(xn, w_ref[...], preferred_element_type=jnp.float32)
    h2_ref[...] = h2
    hs2 = h2 * dinv
    hs2_ref[0] = hs2[:, :HH]
    hs2_ref[1] = hs2[:, HH:]


def _tc_final_body(acc_ref, h_ref, deg_ref, b_ref, g_ref, bb_ref,
                   mw1_ref, mb1_ref, mw2_ref, mb2_ref, y_ref):
    dinv = _dinv_from(deg_ref)
    xn = _ln_relu_agg(acc_ref, h_ref, dinv, b_ref, g_ref, bb_ref)
    u = jnp.maximum(
        jnp.dot(xn, mw1_ref[...], preferred_element_type=jnp.float32) + mb1_ref[...],
        0.0,
    )
    y_ref[...] = (
        jnp.dot(u, mw2_ref[...], preferred_element_type=jnp.float32) + mb2_ref[...]
    )


_GRID = N // RB

_z_spec = pl.BlockSpec((RB, H), lambda i: (i, 0))
_w_spec = pl.BlockSpec((H, H), lambda i: (0, 0))
_deg_spec = pl.BlockSpec((NC, RB, DEGW), lambda i: (0, i, 0))
_acc_spec = pl.BlockSpec((NC, RB, HH), lambda i: (0, i, 0))
_vec_spec = pl.BlockSpec((1, H), lambda i: (0, 0))

_nh_struct = jax.ShapeDtypeStruct((N, H), jnp.float32)
_split_struct = jax.ShapeDtypeStruct((NC, N, HH), jnp.float32)
_split_spec = pl.BlockSpec((NC, RB, HH), lambda i: (0, i, 0))

_tc_prep = pl.pallas_call(
    _tc_prep_body,
    grid=(_GRID,),
    in_specs=[_z_spec, _w_spec, _deg_spec],
    out_specs=[_z_spec, _split_spec],
    out_shape=[_nh_struct, _split_struct],
)

_tc_mid = pl.pallas_call(
    _tc_mid_body,
    grid=(_GRID,),
    in_specs=[_acc_spec, _z_spec, _deg_spec, _vec_spec, _vec_spec, _vec_spec, _w_spec],
    out_specs=[_z_spec, _split_spec],
    out_shape=[_nh_struct, _split_struct],
)

_tc_final = pl.pallas_call(
    _tc_final_body,
    grid=(_GRID,),
    in_specs=[_acc_spec, _z_spec, _deg_spec, _vec_spec, _vec_spec, _vec_spec,
              _w_spec, _vec_spec, _w_spec, _vec_spec],
    out_specs=_z_spec,
    out_shape=_nh_struct,
)


@jax.jit
def kernel(z, edge_index, W1, b1, g1, bb1, W2, b2, g2, bb2, mW1, mb1, mW2, mb2):
    dst_d = edge_index[1].reshape(NW, NCH_D, CHUNK)
    src_p = edge_index[0].reshape(NS, NCH_P, CHUNK)
    dst_p = edge_index[1].reshape(NS, NCH_P, CHUNK)

    deg2 = _sc_deg(dst_d)
    h1, hs1 = _tc_prep(z, W1, deg2)
    acc1 = _sc_pass(hs1, src_p, dst_p)
    h2, hs2 = _tc_mid(
        acc1, h1, deg2,
        b1.reshape(1, H), g1.reshape(1, H), bb1.reshape(1, H), W2,
    )
    acc2 = _sc_pass(hs2, src_p, dst_p)

    mW2p = jnp.zeros((H, H), jnp.float32).at[:, :C].set(mW2)
    mb2p = jnp.zeros((1, H), jnp.float32).at[0, :C].set(mb2)
    y = _tc_final(
        acc2, h2, deg2,
        b2.reshape(1, H), g2.reshape(1, H), bb2.reshape(1, H),
        mW1, mb1.reshape(1, H), mW2p, mb2p,
    )
    return y[:, :C]


# speedup vs baseline: 22.9416x; 1.3198x over previous
import jax
import jax.numpy as jnp
from jax import lax
from jax.experimental import pallas as pl
from jax.experimental.pallas import tpu as pltpu
from jax.experimental.pallas import tpu_sc as plsc

N = 10000
D = 128
H = 128
C = 3
E = 320000

NC = 2
NS = 16
NW = NC * NS
HH = H // NC
CHUNK = 80

NBUF = 4
NCH_P = 252
EPT = NCH_P * CHUNK
E_PAD = NS * EPT
EPW = E // NW
NCH_D = EPW // CHUNK

NPAD = 10240
RPT = NPAD // NS
ZROWS = 128
DEGW = 16

_mesh = plsc.VectorSubcoreMesh(
    core_axis_name="c", subcore_axis_name="s", num_cores=NC, num_subcores=NS
)


def _sc_deg_body(dst_hbm, deg_out, dstv, onesv, zv, deg_sh):
    c = lax.axis_index("c")
    s = lax.axis_index("s")
    wid = s * NC + c

    def fill_zero(i, carry):
        zv[i, :] = jnp.zeros((16,), jnp.float32)
        return carry

    lax.fori_loop(0, RPT, fill_zero, 0)

    def fill_one(i, carry):
        onesv[i, :] = jnp.ones((16,), jnp.float32)
        return carry

    lax.fori_loop(0, CHUNK, fill_one, 0)

    pltpu.sync_copy(zv, deg_sh.at[pl.ds(s * RPT, RPT)])
    plsc.subcore_barrier()

    pltpu.sync_copy(dst_hbm.at[wid], dstv)

    def step(ci, carry):
        pltpu.sync_copy(onesv, deg_sh.at[dstv.at[ci]], add=True)
        return carry

    lax.fori_loop(0, NCH_D, step, 0)
    plsc.subcore_barrier()

    pltpu.sync_copy(
        deg_sh.at[pl.ds(s * RPT, RPT)], deg_out.at[c, pl.ds(s * RPT, RPT)]
    )


_sc_deg = pl.kernel(
    _sc_deg_body,
    out_type=jax.ShapeDtypeStruct((NC, NPAD, DEGW), jnp.float32),
    mesh=_mesh,
    scratch_types=[
        pltpu.VMEM((NCH_D, CHUNK), jnp.int32),
        pltpu.VMEM((CHUNK, DEGW), jnp.float32),
        pltpu.VMEM((RPT, DEGW), jnp.float32),
        pltpu.VMEM_SHARED((NPAD, DEGW), jnp.float32),
    ],
    compiler_params=pltpu.CompilerParams(use_tc_tiling_on_sc=False),
)


def _sc_pass_body(hs_hbm, src_hbm, dst_hbm, acc_out,
                  srcv, dstv, rows0, rows1, rows2, rows3, zb, acc_sh, gsem, ssem):
    rows = (rows0, rows1, rows2, rows3)
    c = lax.axis_index("c")
    s = lax.axis_index("s")

    def fill_zero(i, carry):
        for j in range(HH // 16):
            zb[i, pl.ds(j * 16, 16)] = jnp.zeros((16,), jnp.float32)
        return carry

    lax.fori_loop(0, ZROWS, fill_zero, 0)
    for k in range(RPT // ZROWS):
        pltpu.sync_copy(zb, acc_sh.at[pl.ds(s * RPT + k * ZROWS, ZROWS)])
    plsc.subcore_barrier()

    pltpu.sync_copy(src_hbm.at[s], srcv)
    pltpu.sync_copy(dst_hbm.at[s], dstv)

    def gather(ci, buf):
        return pltpu.make_async_copy(hs_hbm.at[c].at[srcv.at[ci]], buf, gsem)

    def scatter_start(ci, buf):
        pltpu.async_copy(buf, acc_sh.at[dstv.at[ci]], ssem, add=True)

    def scatter_wait(ci, buf):
        pltpu.make_async_copy(buf, acc_sh.at[dstv.at[ci]], ssem).wait()

    for b in range(NBUF - 1):
        gather(b, rows[b]).start()

    def step(g, carry):
        for b in range(NBUF):
            ci = g * NBUF + b
            prev = rows[(b + NBUF - 1) % NBUF]
            gather(ci, rows[b]).wait()
            scatter_start(ci, rows[b])
            if b == 0:
                @pl.when(g >= 1)
                def _():
                    scatter_wait(ci - 1, prev)
                gather(ci + NBUF - 1, prev).start()
            else:
                scatter_wait(ci - 1, prev)

                @pl.when(g + 1 < NCH_P // NBUF)
                def _():
                    gather(ci + NBUF - 1, prev).start()
        return carry

    lax.fori_loop(0, NCH_P // NBUF, step, 0)
    scatter_wait(NCH_P - 1, rows[(NCH_P - 1) % NBUF])
    plsc.subcore_barrier()

    pltpu.sync_copy(
        acc_sh.at[pl.ds(s * RPT, RPT)], acc_out.at[c, pl.ds(s * RPT, RPT)]
    )


_sc_pass = pl.kernel(
    _sc_pass_body,
    out_type=jax.ShapeDtypeStruct((NC, NPAD, HH), jnp.float32),
    mesh=_mesh,
    scratch_types=[
        pltpu.VMEM((NCH_P, CHUNK), jnp.int32),
        pltpu.VMEM((NCH_P, CHUNK), jnp.int32),
        pltpu.VMEM((CHUNK, HH), jnp.float32),
        pltpu.VMEM((CHUNK, HH), jnp.float32),
        pltpu.VMEM((CHUNK, HH), jnp.float32),
        pltpu.VMEM((CHUNK, HH), jnp.float32),
        pltpu.VMEM((ZROWS, HH), jnp.float32),
        pltpu.VMEM_SHARED((NPAD, HH), jnp.float32),
        pltpu.SemaphoreType.DMA,
        pltpu.SemaphoreType.DMA,
    ],
    compiler_params=pltpu.CompilerParams(use_tc_tiling_on_sc=False),
)


RB = 2000


def _dinv_from(deg_ref):
    deg = deg_ref[0, :, 0:1] + deg_ref[1, :, 0:1] + 1.0
    return lax.rsqrt(deg)


def _tc_prep_body(z_ref, w_ref, deg_ref, h_ref, hs_ref):
    dinv = _dinv_from(deg_ref)
    h = jnp.dot(z_ref[...], w_ref[...], preferred_element_type=jnp.float32)
    h_ref[...] = h
    hs = h * dinv
    hs_ref[0] = hs[:, :HH]
    hs_ref[1] = hs[:, HH:]


def _ln_relu_agg(acc_ref, h_ref, dinv, b_ref, g_ref, bb_ref):
    acc = jnp.concatenate([acc_ref[0], acc_ref[1]], axis=-1)
    agg = acc * dinv + h_ref[...] * (dinv * dinv) + b_ref[...]
    x = jnp.maximum(agg, 0.0)
    mu = jnp.mean(x, axis=-1, keepdims=True)
    xc = x - mu
    var = jnp.mean(xc * xc, axis=-1, keepdims=True)
    return xc * lax.rsqrt(var + 1e-5) * g_ref[...] + bb_ref[...]


def _tc_mid_body(acc_ref, h_ref, deg_ref, b_ref, g_ref, bb_ref, w_ref, h2_ref, hs2_ref):
    dinv = _dinv_from(deg_ref)
    xn = _ln_relu_agg(acc_ref, h_ref, dinv, b_ref, g_ref, bb_ref)
    h2 = jnp.dot(xn, w_ref[...], preferred_element_type=jnp.float32)
    h2_ref[...] = h2
    hs2 = h2 * dinv
    hs2_ref[0] = hs2[:, :HH]
    hs2_ref[1] = hs2[:, HH:]


def _tc_final_body(acc_ref, h_ref, deg_ref, b_ref, g_ref, bb_ref,
                   mw1_ref, mb1_ref, mw2_ref, mb2_ref, y_ref):
    dinv = _dinv_from(deg_ref)
    xn = _ln_relu_agg(acc_ref, h_ref, dinv, b_ref, g_ref, bb_ref)
    u = jnp.maximum(
        jnp.dot(xn, mw1_ref[...], preferred_element_type=jnp.float32) + mb1_ref[...],
        0.0,
    )
    y_ref[...] = (
        jnp.dot(u, mw2_ref[...], preferred_element_type=jnp.float32) + mb2_ref[...]
    )


_GRID = N // RB

_z_spec = pl.BlockSpec((RB, H), lambda i: (i, 0))
_w_spec = pl.BlockSpec((H, H), lambda i: (0, 0))
_deg_spec = pl.BlockSpec((NC, RB, DEGW), lambda i: (0, i, 0))
_acc_spec = pl.BlockSpec((NC, RB, HH), lambda i: (0, i, 0))
_vec_spec = pl.BlockSpec((1, H), lambda i: (0, 0))

_nh_struct = jax.ShapeDtypeStruct((N, H), jnp.float32)
_split_struct = jax.ShapeDtypeStruct((NC, N, HH), jnp.float32)
_split_spec = pl.BlockSpec((NC, RB, HH), lambda i: (0, i, 0))

_tc_prep = pl.pallas_call(
    _tc_prep_body,
    grid=(_GRID,),
    in_specs=[_z_spec, _w_spec, _deg_spec],
    out_specs=[_z_spec, _split_spec],
    out_shape=[_nh_struct, _split_struct],
)

_tc_mid = pl.pallas_call(
    _tc_mid_body,
    grid=(_GRID,),
    in_specs=[_acc_spec, _z_spec, _deg_spec, _vec_spec, _vec_spec, _vec_spec, _w_spec],
    out_specs=[_z_spec, _split_spec],
    out_shape=[_nh_struct, _split_struct],
)

_tc_final = pl.pallas_call(
    _tc_final_body,
    grid=(_GRID,),
    in_specs=[_acc_spec, _z_spec, _deg_spec, _vec_spec, _vec_spec, _vec_spec,
              _w_spec, _vec_spec, _w_spec, _vec_spec],
    out_specs=_z_spec,
    out_shape=_nh_struct,
)


@jax.jit
def kernel(z, edge_index, W1, b1, g1, bb1, W2, b2, g2, bb2, mW1, mb1, mW2, mb2):
    dst_d = edge_index[1].reshape(NW, NCH_D, CHUNK)
    npadding = E_PAD - E
    src_p = jnp.concatenate(
        [edge_index[0], jnp.zeros((npadding,), jnp.int32)]
    ).reshape(NS, NCH_P, CHUNK)
    dst_p = jnp.concatenate(
        [edge_index[1], jnp.full((npadding,), NPAD - 1, jnp.int32)]
    ).reshape(NS, NCH_P, CHUNK)

    deg2 = _sc_deg(dst_d)
    h1, hs1 = _tc_prep(z, W1, deg2)
    acc1 = _sc_pass(hs1, src_p, dst_p)
    h2, hs2 = _tc_mid(
        acc1, h1, deg2,
        b1.reshape(1, H), g1.reshape(1, H), bb1.reshape(1, H), W2,
    )
    acc2 = _sc_pass(hs2, src_p, dst_p)

    mW2p = jnp.zeros((H, H), jnp.float32).at[:, :C].set(mW2)
    mb2p = jnp.zeros((1, H), jnp.float32).at[0, :C].set(mb2)
    y = _tc_final(
        acc2, h2, deg2,
        b2.reshape(1, H), g2.reshape(1, H), bb2.reshape(1, H),
        mW1, mb1.reshape(1, H), mW2p, mb2p,
    )
    return y[:, :C]
